# baseline (speedup 1.0000x reference)
import functools

import jax
import jax.numpy as jnp
from jax import lax
from jax.experimental import pallas as pl
from jax.experimental.pallas import tpu as pltpu
from jax.experimental.pallas import tpu_sc as plsc

VOCAB = 100000
D = 128
L = 200
B = 1024
N = B * L

NC = 2
NS = 16
NW = NC * NS
PER_W = N // NW
CHUNK = 64
NCHUNK = PER_W // CHUNK
NSLOT = 4
GRP = 16
COMB_PAD = 608
ROWS_PER_TILE = COMB_PAD // NS

_mesh = plsc.VectorSubcoreMesh(core_axis_name="c", subcore_axis_name="s")


@functools.partial(
    pl.kernel,
    out_type=jax.ShapeDtypeStruct((N, D), jnp.float32),
    mesh=_mesh,
    compiler_params=pltpu.CompilerParams(use_tc_tiling_on_sc=True),
    scratch_types=(
        [pltpu.VMEM_SHARED((COMB_PAD, D), jnp.float32)]
        + [pltpu.VMEM((3 * D,), jnp.float32)]
        + [pltpu.VMEM((D,), jnp.float32)] * 2
        + [pltpu.VMEM((CHUNK,), jnp.int32)] * NSLOT
        + [pltpu.VMEM((CHUNK,), jnp.int32)] * NSLOT
        + [pltpu.VMEM((CHUNK,), jnp.int32)] * NSLOT
        + [pltpu.VMEM((CHUNK, D), jnp.float32)] * NSLOT
        + [pltpu.VMEM((CHUNK, D), jnp.float32)] * NSLOT
        + [pltpu.SemaphoreType.DMA]
        + [pltpu.SemaphoreType.DMA] * NSLOT
        + [pltpu.SemaphoreType.DMA] * NSLOT
        + [pltpu.SemaphoreType.DMA] * NSLOT
    ),
)
def _emb_kernel(x_hbm, seg_hbm, tok_hbm, segtab_hbm, pe_hbm, out_hbm,
                comb_sh, segtab_v, rowa, rowb, *rest):
    idx = rest[0:NSLOT]
    sgv = rest[NSLOT:2 * NSLOT]
    cidx = rest[2 * NSLOT:3 * NSLOT]
    tok = rest[3 * NSLOT:4 * NSLOT]
    comb = rest[4 * NSLOT:5 * NSLOT]
    isem = rest[5 * NSLOT]
    gsem = rest[5 * NSLOT + 1:6 * NSLOT + 1]
    csem = rest[6 * NSLOT + 1:7 * NSLOT + 1]
    ssem = rest[7 * NSLOT + 1:8 * NSLOT + 1]

    cid = lax.axis_index("c")
    tid = lax.axis_index("s")
    wid = tid * NC + cid

    pltpu.sync_copy(segtab_hbm, segtab_v)

    def build_row(j, carry):
        row = tid * ROWS_PER_TILE + j
        s = jnp.minimum(row // L, 2)
        ll = jnp.minimum(row - s * L, L - 1)
        pltpu.sync_copy(pe_hbm.at[pl.ds(ll * D, D)], rowb)
        for g in range(D // GRP):
            o = g * GRP
            rowa[pl.ds(o, GRP)] = (
                rowb[pl.ds(o, GRP)] + segtab_v[pl.ds(s * D + o, GRP)]
            )
        pltpu.sync_copy(rowa, comb_sh.at[row])
        return carry

    lax.fori_loop(0, ROWS_PER_TILE, build_row, 0)
    plsc.subcore_barrier()

    base_w = wid * PER_W

    def stage_i(c, p):
        base = base_w + c * CHUNK
        pltpu.make_async_copy(x_hbm.at[pl.ds(base, CHUNK)], idx[p], isem).start()
        pltpu.make_async_copy(seg_hbm.at[pl.ds(base, CHUNK)], sgv[p], isem).start()

    def stage_g(c, p, wait_store):
        base = base_w + c * CHUNK
        pltpu.make_async_copy(x_hbm.at[pl.ds(base, CHUNK)], idx[p], isem).wait()
        pltpu.make_async_copy(seg_hbm.at[pl.ds(base, CHUNK)], sgv[p], isem).wait()
        if wait_store:
            pltpu.make_async_copy(tok[p], out_hbm.at[pl.ds(0, CHUNK)],
                                  ssem[p]).wait()
        for g in range(CHUNK // GRP):
            o = g * GRP
            fi = base + o + lax.iota(jnp.int32, GRP)
            cidx[p][pl.ds(o, GRP)] = (
                sgv[p][pl.ds(o, GRP)] * L + lax.rem(fi, L)
            )
        for q in range(4):
            sub = CHUNK // 4
            pltpu.make_async_copy(
                tok_hbm.at[idx[p].at[pl.ds(q * sub, sub)]],
                tok[p].at[pl.ds(q * sub, sub), :], gsem[p]).start()
        pltpu.make_async_copy(comb_sh.at[cidx[p]], comb[p], csem[p]).start()

    def stage_f(c, p):
        base = base_w + c * CHUNK
        for q in range(4):
            sub = CHUNK // 4
            pltpu.make_async_copy(
                tok_hbm.at[idx[p].at[pl.ds(q * sub, sub)]],
                tok[p].at[pl.ds(q * sub, sub), :], gsem[p]).wait()
        pltpu.make_async_copy(comb_sh.at[cidx[p]], comb[p], csem[p]).wait()

        @plsc.parallel_loop(0, CHUNK, step=1, unroll=4)
        def add_row(r):
            for g in range(D // GRP):
                o = g * GRP
                plsc.addupdate(tok[p].at[r, pl.ds(o, GRP)],
                               comb[p][r, pl.ds(o, GRP)])

        pltpu.make_async_copy(tok[p], out_hbm.at[pl.ds(base, CHUNK)],
                              ssem[p]).start()

    stage_i(0, 0)
    stage_g(0, 0, False)
    stage_i(1, 1)
    stage_g(1, 1, False)
    stage_i(2, 2)
    for j, c in enumerate(range(4)):
        stage_g(c + 2, (c + 2) % NSLOT, (c + 2) >= NSLOT)
        stage_f(c, c % NSLOT)
        stage_i(c + 3, (c + 3) % NSLOT)

    def body(cc, carry):
        c0 = 4 * cc
        for j in range(4):
            c = c0 + j
            stage_g(c + 2, (j + 2) % NSLOT, True)
            stage_f(c, j)
            stage_i(c + 3, (j + 3) % NSLOT)
        return carry

    lax.fori_loop(1, NCHUNK // 4 - 1, body, 0)
    c0 = NCHUNK - 4
    stage_g(c0 + 2, (c0 + 2) % NSLOT, True)
    stage_f(c0, c0 % NSLOT)
    stage_i(c0 + 3, (c0 + 3) % NSLOT)
    stage_g(c0 + 3, (c0 + 3) % NSLOT, True)
    stage_f(c0 + 1, (c0 + 1) % NSLOT)
    stage_f(c0 + 2, (c0 + 2) % NSLOT)
    stage_f(c0 + 3, (c0 + 3) % NSLOT)
    for p in range(NSLOT):
        pltpu.make_async_copy(tok[p], out_hbm.at[pl.ds(0, CHUNK)],
                              ssem[p]).wait()


def kernel(x, seg, tok_table, seg_table, pe):
    x_flat = x.reshape(-1).astype(jnp.int32)
    seg_flat = seg.reshape(-1).astype(jnp.int32)
    pe_flat = pe[0, :L, :].reshape(-1).astype(jnp.float32)
    segtab_flat = seg_table.reshape(-1).astype(jnp.float32)
    out = _emb_kernel(x_flat, seg_flat, tok_table, segtab_flat, pe_flat)
    return out.reshape(B, L, D)


# speedup vs baseline: 1.1319x; 1.1319x over previous
import functools

import jax
import jax.numpy as jnp
from jax import lax
from jax.experimental import pallas as pl
from jax.experimental.pallas import tpu as pltpu
from jax.experimental.pallas import tpu_sc as plsc

VOCAB = 100000
D = 128
L = 200
B = 1024
N = B * L

NC = 2
NS = 16
NW = NC * NS
PER_W = N // NW
CHUNK = 64
NCHUNK = PER_W // CHUNK
NSLOT = 4
GRP = 16
COMB_PAD = 608
ROWS_PER_TILE = COMB_PAD // NS

_mesh = plsc.VectorSubcoreMesh(core_axis_name="c", subcore_axis_name="s")


@functools.partial(
    pl.kernel,
    out_type=jax.ShapeDtypeStruct((N, D), jnp.float32),
    mesh=_mesh,
    scratch_types=(
        [pltpu.VMEM_SHARED((COMB_PAD, D), jnp.float32)]
        + [pltpu.VMEM((3 * D,), jnp.float32)]
        + [pltpu.VMEM((L * D,), jnp.float32)]
        + [pltpu.VMEM((ROWS_PER_TILE, D), jnp.float32)]
        + [pltpu.VMEM((CHUNK,), jnp.int32)] * NSLOT
        + [pltpu.VMEM((CHUNK,), jnp.int32)] * NSLOT
        + [pltpu.VMEM((CHUNK,), jnp.int32)] * NSLOT
        + [pltpu.VMEM((CHUNK, D), jnp.float32)] * NSLOT
        + [pltpu.VMEM((CHUNK, D), jnp.float32)] * NSLOT
        + [pltpu.SemaphoreType.DMA]
        + [pltpu.SemaphoreType.DMA]
        + [pltpu.SemaphoreType.DMA] * NSLOT
        + [pltpu.SemaphoreType.DMA] * NSLOT
        + [pltpu.SemaphoreType.DMA] * NSLOT
    ),
)
def _emb_kernel(x_hbm, seg_hbm, tok_hbm, segtab_hbm, pe_hbm, out_hbm,
                comb_sh, segtab_v, pe_v, bld, *rest):
    idx = rest[0:NSLOT]
    sgv = rest[NSLOT:2 * NSLOT]
    cidx = rest[2 * NSLOT:3 * NSLOT]
    tok = rest[3 * NSLOT:4 * NSLOT]
    comb = rest[4 * NSLOT:5 * NSLOT]
    psem = rest[5 * NSLOT]
    isem = rest[5 * NSLOT + 1]
    gsem = rest[5 * NSLOT + 2:6 * NSLOT + 2]
    csem = rest[6 * NSLOT + 2:7 * NSLOT + 2]
    ssem = rest[7 * NSLOT + 2:8 * NSLOT + 2]

    cid = lax.axis_index("c")
    tid = lax.axis_index("s")
    wid = tid * NC + cid
    base_w = wid * PER_W

    def stage_i(c, p):
        base = base_w + c * CHUNK
        pltpu.make_async_copy(x_hbm.at[pl.ds(base, CHUNK)], idx[p], isem).start()
        pltpu.make_async_copy(seg_hbm.at[pl.ds(base, CHUNK)], sgv[p], isem).start()

    def comb_launch(p):
        pltpu.make_async_copy(comb_sh.at[cidx[p]], comb[p], csem[p]).start()

    def stage_g(c, p, wait_store, with_comb=True):
        base = base_w + c * CHUNK
        pltpu.make_async_copy(x_hbm.at[pl.ds(base, CHUNK)], idx[p], isem).wait()
        pltpu.make_async_copy(seg_hbm.at[pl.ds(base, CHUNK)], sgv[p], isem).wait()
        if wait_store:
            pltpu.make_async_copy(tok[p], out_hbm.at[pl.ds(0, CHUNK)],
                                  ssem[p]).wait()
        for g in range(CHUNK // GRP):
            o = g * GRP
            fi = base + o + lax.iota(jnp.int32, GRP)
            cidx[p][pl.ds(o, GRP)] = (
                sgv[p][pl.ds(o, GRP)] * L + lax.rem(fi, L)
            )
        pltpu.make_async_copy(tok_hbm.at[idx[p]], tok[p], gsem[p]).start()
        if with_comb:
            comb_launch(p)

    def stage_f(c, p):
        base = base_w + c * CHUNK
        pltpu.make_async_copy(tok_hbm.at[idx[p]], tok[p], gsem[p]).wait()
        pltpu.make_async_copy(comb_sh.at[cidx[p]], comb[p], csem[p]).wait()

        @plsc.parallel_loop(0, CHUNK, step=1, unroll=4)
        def add_row(r):
            for g in range(D // GRP):
                o = g * GRP
                plsc.addupdate(tok[p].at[r, pl.ds(o, GRP)],
                               comb[p][r, pl.ds(o, GRP)])

        pltpu.make_async_copy(tok[p], out_hbm.at[pl.ds(base, CHUNK)],
                              ssem[p]).start()

    pe_cp = pltpu.make_async_copy(pe_hbm, pe_v, psem)
    pe_cp.start()
    pltpu.sync_copy(segtab_hbm, segtab_v)
    stage_i(0, 0)
    stage_g(0, 0, False, with_comb=False)
    stage_i(1, 1)
    stage_g(1, 1, False, with_comb=False)
    stage_i(2, 2)

    pe_cp.wait()

    @plsc.parallel_loop(0, ROWS_PER_TILE, step=1, unroll=2)
    def build_row(j):
        row = tid * ROWS_PER_TILE + j
        s = jnp.minimum(row // L, 2)
        ll = jnp.minimum(row - s * L, L - 1)
        for g in range(D // GRP):
            o = g * GRP
            bld[j, pl.ds(o, GRP)] = (
                pe_v[pl.ds(ll * D + o, GRP)] + segtab_v[pl.ds(s * D + o, GRP)]
            )

    pltpu.sync_copy(
        bld, comb_sh.at[pl.ds(tid * ROWS_PER_TILE, ROWS_PER_TILE), :])
    plsc.subcore_barrier()
    comb_launch(0)
    comb_launch(1)

    for c in range(4):
        stage_g(c + 2, (c + 2) % NSLOT, (c + 2) >= NSLOT)
        stage_f(c, c % NSLOT)
        stage_i(c + 3, (c + 3) % NSLOT)

    def body(cc, carry):
        c0 = 4 * cc
        for j in range(4):
            c = c0 + j
            stage_g(c + 2, (j + 2) % NSLOT, True)
            stage_f(c, j)
            stage_i(c + 3, (j + 3) % NSLOT)
        return carry

    lax.fori_loop(1, NCHUNK // 4 - 1, body, 0)
    c0 = NCHUNK - 4
    stage_g(c0 + 2, (c0 + 2) % NSLOT, True)
    stage_f(c0, c0 % NSLOT)
    stage_i(c0 + 3, (c0 + 3) % NSLOT)
    stage_g(c0 + 3, (c0 + 3) % NSLOT, True)
    stage_f(c0 + 1, (c0 + 1) % NSLOT)
    stage_f(c0 + 2, (c0 + 2) % NSLOT)
    stage_f(c0 + 3, (c0 + 3) % NSLOT)
    for p in range(NSLOT):
        pltpu.make_async_copy(tok[p], out_hbm.at[pl.ds(0, CHUNK)],
                              ssem[p]).wait()


def kernel(x, seg, tok_table, seg_table, pe):
    x_flat = x.reshape(-1).astype(jnp.int32)
    seg_flat = seg.reshape(-1).astype(jnp.int32)
    pe_flat = pe[0, :L, :].reshape(-1).astype(jnp.float32)
    segtab_flat = seg_table.reshape(-1).astype(jnp.float32)
    out = _emb_kernel(x_flat, seg_flat, tok_table, segtab_flat, pe_flat)
    return out.reshape(B, L, D)


# speedup vs baseline: 1.1354x; 1.0031x over previous
import functools

import jax
import jax.numpy as jnp
from jax import lax
from jax.experimental import pallas as pl
from jax.experimental.pallas import tpu as pltpu
from jax.experimental.pallas import tpu_sc as plsc

VOCAB = 100000
D = 128
L = 200
B = 1024
N = B * L

NC = 2
NS = 16
NW = NC * NS
PER_W = N // NW
CHUNK = 64
NCHUNK = PER_W // CHUNK
NSLOT = 4
GRP = 16
COMB_PAD = 608
ROWS_PER_TILE = COMB_PAD // NS

_mesh = plsc.VectorSubcoreMesh(core_axis_name="c", subcore_axis_name="s")


@functools.partial(
    pl.kernel,
    out_type=jax.ShapeDtypeStruct((N, D), jnp.float32),
    mesh=_mesh,
    scratch_types=(
        [pltpu.VMEM_SHARED((COMB_PAD, D), jnp.float32)]
        + [pltpu.VMEM((3 * D,), jnp.float32)]
        + [pltpu.VMEM((L * D,), jnp.float32)]
        + [pltpu.VMEM((ROWS_PER_TILE, D), jnp.float32)]
        + [pltpu.VMEM((CHUNK,), jnp.int32)] * NSLOT
        + [pltpu.VMEM((CHUNK,), jnp.int32)] * NSLOT
        + [pltpu.VMEM((CHUNK,), jnp.int32)] * NSLOT
        + [pltpu.VMEM((CHUNK, D), jnp.float32)] * NSLOT
        + [pltpu.VMEM((CHUNK, D), jnp.float32)] * NSLOT
        + [pltpu.SemaphoreType.DMA]
        + [pltpu.SemaphoreType.DMA]
        + [pltpu.SemaphoreType.DMA] * NSLOT
        + [pltpu.SemaphoreType.DMA] * NSLOT
        + [pltpu.SemaphoreType.DMA] * NSLOT
    ),
)
def _emb_kernel(x_hbm, seg_hbm, tok_hbm, segtab_hbm, pe_hbm, out_hbm,
                comb_sh, segtab_v, pe_v, bld, *rest):
    idx = rest[0:NSLOT]
    sgv = rest[NSLOT:2 * NSLOT]
    cidx = rest[2 * NSLOT:3 * NSLOT]
    tok = rest[3 * NSLOT:4 * NSLOT]
    comb = rest[4 * NSLOT:5 * NSLOT]
    psem = rest[5 * NSLOT]
    isem = rest[5 * NSLOT + 1]
    gsem = rest[5 * NSLOT + 2:6 * NSLOT + 2]
    csem = rest[6 * NSLOT + 2:7 * NSLOT + 2]
    ssem = rest[7 * NSLOT + 2:8 * NSLOT + 2]

    cid = lax.axis_index("c")
    tid = lax.axis_index("s")
    wid = tid * NC + cid
    base_w = wid * PER_W

    def stage_i(c, p):
        base = base_w + c * CHUNK
        pltpu.make_async_copy(x_hbm.at[pl.ds(base, CHUNK)], idx[p], isem).start()
        pltpu.make_async_copy(seg_hbm.at[pl.ds(base, CHUNK)], sgv[p], isem).start()

    def comb_launch(p):
        pltpu.make_async_copy(comb_sh.at[cidx[p]], comb[p], csem[p]).start()

    def stage_g(c, p, wait_store, with_comb=True):
        base = base_w + c * CHUNK
        pltpu.make_async_copy(x_hbm.at[pl.ds(base, CHUNK)], idx[p], isem).wait()
        pltpu.make_async_copy(seg_hbm.at[pl.ds(base, CHUNK)], sgv[p], isem).wait()
        if wait_store:
            pltpu.make_async_copy(tok[p], out_hbm.at[pl.ds(0, CHUNK)],
                                  ssem[p]).wait()
        for g in range(CHUNK // GRP):
            o = g * GRP
            fi = base + o + lax.iota(jnp.int32, GRP)
            cidx[p][pl.ds(o, GRP)] = (
                sgv[p][pl.ds(o, GRP)] * L + lax.rem(fi, L)
            )
        pltpu.make_async_copy(tok_hbm.at[idx[p]], tok[p], gsem[p]).start()
        if with_comb:
            comb_launch(p)

    def stage_f(c, p):
        base = base_w + c * CHUNK
        pltpu.make_async_copy(tok_hbm.at[idx[p]], tok[p], gsem[p]).wait()
        pltpu.make_async_copy(comb_sh.at[cidx[p]], comb[p], csem[p]).wait()

        @plsc.parallel_loop(0, CHUNK, step=1, unroll=4)
        def add_row(r):
            for g in range(D // GRP):
                o = g * GRP
                plsc.addupdate(tok[p].at[r, pl.ds(o, GRP)],
                               comb[p][r, pl.ds(o, GRP)])

        pltpu.make_async_copy(tok[p], out_hbm.at[pl.ds(base, CHUNK)],
                              ssem[p]).start()

    pe_cp = pltpu.make_async_copy(pe_hbm, pe_v, psem)
    pe_cp.start()
    pltpu.sync_copy(segtab_hbm, segtab_v)
    for c in range(NSLOT):
        stage_i(c, c)
        stage_g(c, c, False, with_comb=False)

    pe_cp.wait()

    @plsc.parallel_loop(0, ROWS_PER_TILE, step=1, unroll=2)
    def build_row(j):
        row = tid * ROWS_PER_TILE + j
        s = jnp.minimum(row // L, 2)
        ll = jnp.minimum(row - s * L, L - 1)
        for g in range(D // GRP):
            o = g * GRP
            bld[j, pl.ds(o, GRP)] = (
                pe_v[pl.ds(ll * D + o, GRP)] + segtab_v[pl.ds(s * D + o, GRP)]
            )

    pltpu.sync_copy(
        bld, comb_sh.at[pl.ds(tid * ROWS_PER_TILE, ROWS_PER_TILE), :])
    plsc.subcore_barrier()
    for p in range(NSLOT):
        comb_launch(p)

    stage_f(0, 0)
    stage_f(1, 1)
    stage_i(4, 0)
    stage_g(4, 0, True)
    stage_f(2, 2)
    stage_i(5, 1)
    stage_g(5, 1, True)
    stage_f(3, 3)
    stage_i(6, 2)

    def body(cc, carry):
        c0 = 4 * cc
        for j in range(4):
            c = c0 + j
            stage_g(c + 2, (j + 2) % NSLOT, True)
            stage_f(c, j)
            stage_i(c + 3, (j + 3) % NSLOT)
        return carry

    lax.fori_loop(1, NCHUNK // 4 - 1, body, 0)
    c0 = NCHUNK - 4
    stage_g(c0 + 2, (c0 + 2) % NSLOT, True)
    stage_f(c0, c0 % NSLOT)
    stage_i(c0 + 3, (c0 + 3) % NSLOT)
    stage_g(c0 + 3, (c0 + 3) % NSLOT, True)
    stage_f(c0 + 1, (c0 + 1) % NSLOT)
    stage_f(c0 + 2, (c0 + 2) % NSLOT)
    stage_f(c0 + 3, (c0 + 3) % NSLOT)
    for p in range(NSLOT):
        pltpu.make_async_copy(tok[p], out_hbm.at[pl.ds(0, CHUNK)],
                              ssem[p]).wait()


def kernel(x, seg, tok_table, seg_table, pe):
    x_flat = x.reshape(-1).astype(jnp.int32)
    seg_flat = seg.reshape(-1).astype(jnp.int32)
    pe_flat = pe[0, :L, :].reshape(-1).astype(jnp.float32)
    segtab_flat = seg_table.reshape(-1).astype(jnp.float32)
    out = _emb_kernel(x_flat, seg_flat, tok_table, segtab_flat, pe_flat)
    return out.reshape(B, L, D)
